# Initial kernel scaffold; baseline (speedup 1.0000x reference)
import jax
import jax.numpy as jnp
from jax.experimental import pallas as pl


def kernel(xyz, features, W1, g1, b1, W2, g2, b2, W3, g3, b3):
    raise NotImplementedError("write your pallas kernel here")


# speedup vs baseline: 21.5276x; 21.5276x over previous
import jax
import jax.numpy as jnp
from jax import lax
from jax.experimental import pallas as pl
from jax.experimental.pallas import tpu as pltpu
from jax.experimental.pallas import tpu_sc as plsc

_B, _N, _S, _NS = 4, 16384, 1024, 32
_R2 = 0.25
_CIN = 64
_D = 128
_ROWS = 8
_COLS = _N // _ROWS
_BIGF = float(_N)


def _fps_body(xs_ref, ys_ref, zs_ref, si_ref, nx_ref, dist_ref):
    f32 = jnp.float32
    shp = (_B, _ROWS, _COLS)
    iota = (
        lax.broadcasted_iota(jnp.int32, shp, 1) * _COLS
        + lax.broadcasted_iota(jnp.int32, shp, 2)
    ).astype(f32)
    dist_ref[...] = jnp.full(shp, 1e10, f32)

    def body(i, far):
        onehot = (iota == far).astype(f32)
        xs = xs_ref[...]
        ys = ys_ref[...]
        zs = zs_ref[...]
        cx = jnp.sum(jnp.sum(xs * onehot, axis=1, keepdims=True), axis=2, keepdims=True)
        cy = jnp.sum(jnp.sum(ys * onehot, axis=1, keepdims=True), axis=2, keepdims=True)
        cz = jnp.sum(jnp.sum(zs * onehot, axis=1, keepdims=True), axis=2, keepdims=True)
        si_ref[:, pl.ds(i, 1), :] = far.astype(jnp.int32)
        nx_ref[:, pl.ds(i, 1), :] = jnp.concatenate([cx, cy, cz], axis=2)
        dx = xs - cx
        dy = ys - cy
        dz = zs - cz
        d = (dx * dx + dy * dy) + dz * dz
        dmin = jnp.minimum(dist_ref[...], d)
        dist_ref[...] = dmin
        m = jnp.max(jnp.max(dmin, axis=1, keepdims=True), axis=2, keepdims=True)
        cand = jnp.where(dmin == m, iota, f32(3.0e10))
        far2 = jnp.min(jnp.min(cand, axis=1, keepdims=True), axis=2, keepdims=True)
        return far2

    lax.fori_loop(0, _S, body, jnp.zeros((_B, 1, 1), f32))


def _fps(xyz, interpret=False):
    xs = xyz[..., 0].reshape(_B, _ROWS, _COLS)
    ys = xyz[..., 1].reshape(_B, _ROWS, _COLS)
    zs = xyz[..., 2].reshape(_B, _ROWS, _COLS)
    si, nx = pl.pallas_call(
        _fps_body,
        out_shape=[
            jax.ShapeDtypeStruct((_B, _S, 1), jnp.int32),
            jax.ShapeDtypeStruct((_B, _S, 3), jnp.float32),
        ],
        scratch_shapes=[pltpu.VMEM((_B, _ROWS, _COLS), jnp.float32)],
        interpret=interpret,
    )(xs, ys, zs)
    return si[..., 0], nx


_SB = 64
_CH = 2048


def _ball_body(xt_ref, c_ref, o_ref, cand_ref):
    f32 = jnp.float32
    b = pl.program_id(0)
    c = c_ref[0]
    cx = c[:, 0:1]
    cy = c[:, 1:2]
    cz = c[:, 2:3]

    def chunk(k, _):
        sl = pl.ds(k * _CH, _CH)
        xr = xt_ref[0, 0:1, sl]
        yr = xt_ref[0, 1:2, sl]
        zr = xt_ref[0, 2:3, sl]
        dx = xr - cx
        dy = yr - cy
        dz = zr - cz
        d = (dx * dx + dy * dy) + dz * dz
        idxv = (k * _CH + lax.broadcasted_iota(jnp.int32, (1, _CH), 1)).astype(f32)
        cand_ref[:, sl] = jnp.where(d <= _R2, idxv, f32(_BIGF))
        return 0

    lax.fori_loop(0, _N // _CH, chunk, 0)

    lane32 = lax.broadcasted_iota(jnp.int32, (1, _NS), 1).astype(f32)

    def ext(j, acc):
        candv = cand_ref[...]
        m = jnp.min(candv, axis=1, keepdims=True)
        cand_ref[...] = jnp.where(candv == m, f32(_BIGF), candv)
        return acc + m * (lane32 == j.astype(f32)).astype(f32)

    acc = lax.fori_loop(0, _NS, ext, jnp.zeros((_SB, _NS), f32))
    first = acc[:, 0:1]
    acc = jnp.where(acc == _BIGF, first, acc)
    acc = jnp.where(acc == _BIGF, 0.0, acc)
    o_ref[0] = acc.astype(jnp.int32) + b * _N


def _ball(xyzT, new_xyz, interpret=False):
    return pl.pallas_call(
        _ball_body,
        grid=(_B, _S // _SB),
        in_specs=[
            pl.BlockSpec((1, 3, _N), lambda b, s: (b, 0, 0)),
            pl.BlockSpec((1, _SB, 3), lambda b, s: (b, s, 0)),
        ],
        out_specs=pl.BlockSpec((1, _SB, _NS), lambda b, s: (b, s, 0)),
        out_shape=jax.ShapeDtypeStruct((_B, _S, _NS), jnp.int32),
        scratch_shapes=[pltpu.VMEM((_SB, _N), jnp.float32)],
        interpret=interpret,
    )(xyzT, new_xyz)


_NIDX = _B * _S * _NS
_GW = 128


def _sc_gather(table, idx_flat):
    mesh = plsc.VectorSubcoreMesh(core_axis_name="core", subcore_axis_name="subcore")

    @pl.kernel(
        out_type=jax.ShapeDtypeStruct((_NIDX, _D), jnp.float32),
        mesh=mesh,
    )
    def gather_kernel(x_hbm, i_hbm, o_hbm):
        def body(i_vmem, o_vmem):
            pltpu.sync_copy(x_hbm.at[i_vmem.at[0]], o_vmem)

        pltpu.emit_pipeline(
            body,
            grid=(_NIDX // _GW,),
            in_specs=[pl.BlockSpec((1, _GW), index_map=lambda i: (0, i))],
            out_specs=[pl.BlockSpec((_GW, _D), index_map=lambda i: (i, 0))],
            core_axis_name=("core", "subcore"),
            dimension_semantics=(pltpu.PARALLEL,),
        )(i_hbm, o_hbm)

    return gather_kernel(table, idx_flat)


_SBM = 128
_COUT = 128


def _mlp_body(g_ref, nx_ref, w1_ref, g1_ref, b1_ref, w2_ref, g2_ref, b2_ref,
              w3_ref, g3_ref, b3_ref, o_ref):
    f32 = jnp.float32
    g = g_ref[0]
    g = g - nx_ref[0][:, None, :]
    a = g.reshape(_SBM * _NS, _D)
    dn = (((1,), (1,)), ((), ()))
    h = lax.dot_general(a, w1_ref[...], dn, preferred_element_type=f32)
    h = jnp.maximum(h * g1_ref[...] + b1_ref[...], 0.0)
    h = lax.dot_general(h, w2_ref[...], dn, preferred_element_type=f32)
    h = jnp.maximum(h * g2_ref[...] + b2_ref[...], 0.0)
    h = lax.dot_general(h, w3_ref[...], dn, preferred_element_type=f32)
    h = jnp.maximum(h * g3_ref[...] + b3_ref[...], 0.0)
    p = jnp.max(h.reshape(_SBM, _NS, _COUT), axis=1)
    o_ref[0] = p.T


def _mlp(gath, nxp, w1p, g1, b1, w2, g2, b2, w3, g3, b3, interpret=False):
    full = lambda shape: pl.BlockSpec(shape, lambda b, s: tuple(0 for _ in shape))
    return pl.pallas_call(
        _mlp_body,
        grid=(_B, _S // _SBM),
        in_specs=[
            pl.BlockSpec((1, _SBM, _NS, _D), lambda b, s: (b, s, 0, 0)),
            pl.BlockSpec((1, _SBM, _D), lambda b, s: (b, s, 0)),
            full((64, _D)), full((1, 64)), full((1, 64)),
            full((64, 64)), full((1, 64)), full((1, 64)),
            full((_COUT, 64)), full((1, _COUT)), full((1, _COUT)),
        ],
        out_specs=pl.BlockSpec((1, _COUT, _SBM), lambda b, s: (b, 0, s)),
        out_shape=jax.ShapeDtypeStruct((_B, _COUT, _S), jnp.float32),
        interpret=interpret,
    )(gath, nxp, w1p, g1, b1, w2, g2, b2, w3, g3, b3)


def kernel(xyz, features, W1, g1, b1, W2, g2, b2, W3, g3, b3):
    sample_inds, new_xyz = _fps(xyz)

    xyzT = jnp.transpose(xyz, (0, 2, 1))
    idx = _ball(xyzT, new_xyz)

    pad = _D - 3 - _CIN
    table = jnp.concatenate(
        [xyz, features, jnp.zeros((_B, _N, pad), jnp.float32)], axis=-1
    ).reshape(_B * _N, _D)
    gath = _sc_gather(table, idx.reshape(1, _NIDX)).reshape(_B, _S, _NS, _D)

    nxp = jnp.concatenate(
        [new_xyz, jnp.zeros((_B, _S, _D - 3), jnp.float32)], axis=-1
    )
    w1p = jnp.concatenate([W1, jnp.zeros((64, pad), jnp.float32)], axis=-1)
    new_features = _mlp(
        gath, nxp, w1p,
        g1.reshape(1, 64), b1.reshape(1, 64),
        W2, g2.reshape(1, 64), b2.reshape(1, 64),
        W3, g3.reshape(1, _COUT), b3.reshape(1, _COUT),
    )
    return new_xyz, new_features, sample_inds


# speedup vs baseline: 21.7964x; 1.0125x over previous
import jax
import jax.numpy as jnp
from jax import lax
from jax.experimental import pallas as pl
from jax.experimental.pallas import tpu as pltpu
from jax.experimental.pallas import tpu_sc as plsc

_B, _N, _S, _NS = 4, 16384, 1024, 32
_R2 = 0.25
_CIN = 64
_D = 128
_ROWS = 8
_COLS = _N // _ROWS
_BIGF = float(_N)


def _fps_body(xs_ref, ys_ref, zs_ref, si_ref, nx_ref, dist_ref):
    f32 = jnp.float32
    shp = (_B, _ROWS, _COLS)
    iota = (
        lax.broadcasted_iota(jnp.int32, shp, 1) * _COLS
        + lax.broadcasted_iota(jnp.int32, shp, 2)
    ).astype(f32)
    dist_ref[...] = jnp.full(shp, 1e10, f32)

    def body(i, far):
        onehot = (iota == far).astype(f32)
        xs = xs_ref[...]
        ys = ys_ref[...]
        zs = zs_ref[...]
        cx = jnp.sum(jnp.sum(xs * onehot, axis=1, keepdims=True), axis=2, keepdims=True)
        cy = jnp.sum(jnp.sum(ys * onehot, axis=1, keepdims=True), axis=2, keepdims=True)
        cz = jnp.sum(jnp.sum(zs * onehot, axis=1, keepdims=True), axis=2, keepdims=True)
        si_ref[:, pl.ds(i, 1), :] = far.astype(jnp.int32)
        nx_ref[:, pl.ds(i, 1), :] = jnp.concatenate([cx, cy, cz], axis=2)
        dx = xs - cx
        dy = ys - cy
        dz = zs - cz
        d = (dx * dx + dy * dy) + dz * dz
        dmin = jnp.minimum(dist_ref[...], d)
        dist_ref[...] = dmin
        m = jnp.max(jnp.max(dmin, axis=1, keepdims=True), axis=2, keepdims=True)
        cand = jnp.where(dmin == m, iota, f32(3.0e10))
        far2 = jnp.min(jnp.min(cand, axis=1, keepdims=True), axis=2, keepdims=True)
        return far2

    lax.fori_loop(0, _S, body, jnp.zeros((_B, 1, 1), f32))


def _fps(xyz, interpret=False):
    xs = xyz[..., 0].reshape(_B, _ROWS, _COLS)
    ys = xyz[..., 1].reshape(_B, _ROWS, _COLS)
    zs = xyz[..., 2].reshape(_B, _ROWS, _COLS)
    si, nx = pl.pallas_call(
        _fps_body,
        out_shape=[
            jax.ShapeDtypeStruct((_B, _S, 1), jnp.int32),
            jax.ShapeDtypeStruct((_B, _S, 3), jnp.float32),
        ],
        scratch_shapes=[pltpu.VMEM((_B, _ROWS, _COLS), jnp.float32)],
        interpret=interpret,
    )(xs, ys, zs)
    return si[..., 0], nx


_SB = 64
_CH = 4096


_NW = _N // 32


def _ball_body(xt_ref, c_ref, w_ref, o_ref, words_ref):
    f32 = jnp.float32
    i32 = jnp.int32
    b = pl.program_id(0)
    c = c_ref[0]
    cx = c[:, 0:1]
    cy = c[:, 1:2]
    cz = c[:, 2:3]

    def chunk(k, _):
        sl = pl.ds(k * _CH, _CH)
        xr = xt_ref[0, 0:1, sl]
        yr = xt_ref[0, 1:2, sl]
        zr = xt_ref[0, 2:3, sl]
        dx = xr - cx
        dy = yr - cy
        dz = zr - cz
        d = (dx * dx + dy * dy) + dz * dz
        bits = jnp.where(d <= _R2, w_ref[0:1, sl], i32(0))
        words = jnp.sum(bits.reshape(_SB, _CH // 32, 32), axis=2)
        words_ref[:, pl.ds(k * (_CH // 32), _CH // 32)] = words
        return 0

    lax.fori_loop(0, _N // _CH, chunk, 0)

    lane32 = lax.broadcasted_iota(i32, (1, _NS), 1).astype(f32)
    wiota = lax.broadcasted_iota(i32, (1, _NW), 1).astype(f32)

    def ext(j, acc):
        w = words_ref[...]
        nz = w != 0
        cw = jnp.min(jnp.where(nz, wiota, f32(_NW)), axis=1, keepdims=True)
        oh = (wiota == cw).astype(i32)
        wsel = jnp.sum(w * oh, axis=1, keepdims=True)
        lsb = wsel & (-wsel)
        lf = jnp.abs(lsb.astype(f32))
        bit = (lax.bitcast_convert_type(lf, i32) >> 23) - 127
        idxf = cw * f32(32.0) + bit.astype(f32)
        idxf = jnp.where(cw < _NW, idxf, f32(_BIGF))
        words_ref[...] = w - oh * lsb
        return acc + idxf * (lane32 == j.astype(f32)).astype(f32)

    acc = lax.fori_loop(0, _NS, ext, jnp.zeros((_SB, _NS), f32))
    first = acc[:, 0:1]
    acc = jnp.where(acc == _BIGF, first, acc)
    acc = jnp.where(acc == _BIGF, 0.0, acc)
    o_ref[0] = acc.astype(i32) + b * _N


def _ball(xyzT, new_xyz, bitw, interpret=False):
    return pl.pallas_call(
        _ball_body,
        grid=(_B, _S // _SB),
        in_specs=[
            pl.BlockSpec((1, 3, _N), lambda b, s: (b, 0, 0)),
            pl.BlockSpec((1, _SB, 3), lambda b, s: (b, s, 0)),
            pl.BlockSpec((1, _N), lambda b, s: (0, 0)),
        ],
        out_specs=pl.BlockSpec((1, _SB, _NS), lambda b, s: (b, s, 0)),
        out_shape=jax.ShapeDtypeStruct((_B, _S, _NS), jnp.int32),
        scratch_shapes=[pltpu.VMEM((_SB, _NW), jnp.int32)],
        interpret=interpret,
    )(xyzT, new_xyz, bitw)


_NIDX = _B * _S * _NS
_GW = 128


def _sc_gather(table, idx_flat):
    mesh = plsc.VectorSubcoreMesh(core_axis_name="core", subcore_axis_name="subcore")

    @pl.kernel(
        out_type=jax.ShapeDtypeStruct((_NIDX, _D), jnp.float32),
        mesh=mesh,
    )
    def gather_kernel(x_hbm, i_hbm, o_hbm):
        def body(i_vmem, o_vmem):
            pltpu.sync_copy(x_hbm.at[i_vmem.at[0]], o_vmem)

        pltpu.emit_pipeline(
            body,
            grid=(_NIDX // _GW,),
            in_specs=[pl.BlockSpec((1, _GW), index_map=lambda i: (0, i))],
            out_specs=[pl.BlockSpec((_GW, _D), index_map=lambda i: (i, 0))],
            core_axis_name=("core", "subcore"),
            dimension_semantics=(pltpu.PARALLEL,),
        )(i_hbm, o_hbm)

    return gather_kernel(table, idx_flat)


_SBM = 128
_COUT = 128


def _mlp_body(g_ref, nx_ref, w1_ref, g1_ref, b1_ref, w2_ref, g2_ref, b2_ref,
              w3_ref, g3_ref, b3_ref, o_ref):
    f32 = jnp.float32
    g = g_ref[0]
    g = g - nx_ref[0][:, None, :]
    a = g.reshape(_SBM * _NS, _D)
    dn = (((1,), (1,)), ((), ()))
    h = lax.dot_general(a, w1_ref[...], dn, preferred_element_type=f32)
    h = jnp.maximum(h * g1_ref[...] + b1_ref[...], 0.0)
    h = lax.dot_general(h, w2_ref[...], dn, preferred_element_type=f32)
    h = jnp.maximum(h * g2_ref[...] + b2_ref[...], 0.0)
    h = lax.dot_general(h, w3_ref[...], dn, preferred_element_type=f32)
    h = jnp.maximum(h * g3_ref[...] + b3_ref[...], 0.0)
    p = jnp.max(h.reshape(_SBM, _NS, _COUT), axis=1)
    o_ref[0] = p.T


def _mlp(gath, nxp, w1p, g1, b1, w2, g2, b2, w3, g3, b3, interpret=False):
    full = lambda shape: pl.BlockSpec(shape, lambda b, s: tuple(0 for _ in shape))
    return pl.pallas_call(
        _mlp_body,
        grid=(_B, _S // _SBM),
        in_specs=[
            pl.BlockSpec((1, _SBM, _NS, _D), lambda b, s: (b, s, 0, 0)),
            pl.BlockSpec((1, _SBM, _D), lambda b, s: (b, s, 0)),
            full((64, _D)), full((1, 64)), full((1, 64)),
            full((64, 64)), full((1, 64)), full((1, 64)),
            full((_COUT, 64)), full((1, _COUT)), full((1, _COUT)),
        ],
        out_specs=pl.BlockSpec((1, _COUT, _SBM), lambda b, s: (b, 0, s)),
        out_shape=jax.ShapeDtypeStruct((_B, _COUT, _S), jnp.float32),
        interpret=interpret,
    )(gath, nxp, w1p, g1, b1, w2, g2, b2, w3, g3, b3)


def kernel(xyz, features, W1, g1, b1, W2, g2, b2, W3, g3, b3):
    sample_inds, new_xyz = _fps(xyz)

    xyzT = jnp.transpose(xyz, (0, 2, 1))
    bitw = jnp.left_shift(jnp.int32(1), jnp.arange(_N, dtype=jnp.int32) % 32)
    idx = _ball(xyzT, new_xyz, bitw.reshape(1, _N))

    pad = _D - 3 - _CIN
    table = jnp.concatenate(
        [xyz, features, jnp.zeros((_B, _N, pad), jnp.float32)], axis=-1
    ).reshape(_B * _N, _D)
    gath = _sc_gather(table, idx.reshape(1, _NIDX)).reshape(_B, _S, _NS, _D)

    nxp = jnp.concatenate(
        [new_xyz, jnp.zeros((_B, _S, _D - 3), jnp.float32)], axis=-1
    )
    w1p = jnp.concatenate([W1, jnp.zeros((64, pad), jnp.float32)], axis=-1)
    new_features = _mlp(
        gath, nxp, w1p,
        g1.reshape(1, 64), b1.reshape(1, 64),
        W2, g2.reshape(1, 64), b2.reshape(1, 64),
        W3, g3.reshape(1, _COUT), b3.reshape(1, _COUT),
    )
    return new_xyz, new_features, sample_inds


# speedup vs baseline: 31.4603x; 1.4434x over previous
import jax
import jax.numpy as jnp
from jax import lax
from jax.experimental import pallas as pl
from jax.experimental.pallas import tpu as pltpu
from jax.experimental.pallas import tpu_sc as plsc

_B, _N, _S, _NS = 4, 16384, 1024, 32
_R2 = 0.25
_CIN = 64
_D = 128
_ROWS = 8
_COLS = _N // _ROWS
_BIGF = float(_N)


def _fps_body(xs_ref, ys_ref, zs_ref, si_ref, nx_ref, dist_ref):
    f32 = jnp.float32
    shp = (_B, _ROWS, _COLS)
    iota = (
        lax.broadcasted_iota(jnp.int32, shp, 1) * _COLS
        + lax.broadcasted_iota(jnp.int32, shp, 2)
    ).astype(f32)
    dist_ref[...] = jnp.full(shp, 1e10, f32)

    def body(i, far):
        onehot = (iota == far).astype(f32)
        xs = xs_ref[...]
        ys = ys_ref[...]
        zs = zs_ref[...]
        cx = jnp.sum(jnp.sum(xs * onehot, axis=1, keepdims=True), axis=2, keepdims=True)
        cy = jnp.sum(jnp.sum(ys * onehot, axis=1, keepdims=True), axis=2, keepdims=True)
        cz = jnp.sum(jnp.sum(zs * onehot, axis=1, keepdims=True), axis=2, keepdims=True)
        si_ref[:, pl.ds(i, 1), :] = far.astype(jnp.int32)
        nx_ref[:, pl.ds(i, 1), :] = jnp.concatenate([cx, cy, cz], axis=2)
        dx = xs - cx
        dy = ys - cy
        dz = zs - cz
        d = (dx * dx + dy * dy) + dz * dz
        dmin = jnp.minimum(dist_ref[...], d)
        dist_ref[...] = dmin
        m = jnp.max(jnp.max(dmin, axis=1, keepdims=True), axis=2, keepdims=True)
        cand = jnp.where(dmin == m, iota, f32(3.0e10))
        far2 = jnp.min(jnp.min(cand, axis=1, keepdims=True), axis=2, keepdims=True)
        return far2

    lax.fori_loop(0, _S, body, jnp.zeros((_B, 1, 1), f32))


def _fps(xyz, interpret=False):
    xs = xyz[..., 0].reshape(_B, _ROWS, _COLS)
    ys = xyz[..., 1].reshape(_B, _ROWS, _COLS)
    zs = xyz[..., 2].reshape(_B, _ROWS, _COLS)
    si, nx = pl.pallas_call(
        _fps_body,
        out_shape=[
            jax.ShapeDtypeStruct((_B, _S, 1), jnp.int32),
            jax.ShapeDtypeStruct((_B, _S, 3), jnp.float32),
        ],
        scratch_shapes=[pltpu.VMEM((_B, _ROWS, _COLS), jnp.float32)],
        interpret=interpret,
    )(xs, ys, zs)
    return si[..., 0], nx


_SB = 64
_CH = 4096


_NW = _N // 32


def _ball_body(xt_ref, c_ref, plo_ref, phi_ref, o_ref, words_ref):
    f32 = jnp.float32
    i32 = jnp.int32
    b = pl.program_id(0)
    c = c_ref[0]
    cx = c[:, 0:1]
    cy = c[:, 1:2]
    cz = c[:, 2:3]

    dn = (((1,), (0,)), ((), ()))

    def chunk(k, _):
        sl = pl.ds(k * _CH, _CH)
        xr = xt_ref[0, 0:1, sl]
        yr = xt_ref[0, 1:2, sl]
        zr = xt_ref[0, 2:3, sl]
        dx = xr - cx
        dy = yr - cy
        dz = zr - cz
        d = (dx * dx + dy * dy) + dz * dz
        bitsb = (d <= _R2).astype(jnp.bfloat16)
        wlo = lax.dot_general(bitsb, plo_ref[...], dn, preferred_element_type=f32)
        whi = lax.dot_general(bitsb, phi_ref[...], dn, preferred_element_type=f32)
        words = wlo.astype(i32) + (whi.astype(i32) << 16)
        words_ref[:, pl.ds(k * (_CH // 32), _CH // 32)] = words
        return 0

    lax.fori_loop(0, _N // _CH, chunk, 0)

    lane32 = lax.broadcasted_iota(i32, (1, _NS), 1).astype(f32)
    wiota = lax.broadcasted_iota(i32, (1, _NW), 1).astype(f32)

    def ext(j, acc):
        w = words_ref[...]
        nz = w != 0
        cw = jnp.min(jnp.where(nz, wiota, f32(_NW)), axis=1, keepdims=True)
        oh = (wiota == cw).astype(i32)
        wsel = jnp.sum(w * oh, axis=1, keepdims=True)
        lsb = wsel & (-wsel)
        lf = jnp.abs(lsb.astype(f32))
        bit = (lax.bitcast_convert_type(lf, i32) >> 23) - 127
        idxf = cw * f32(32.0) + bit.astype(f32)
        idxf = jnp.where(cw < _NW, idxf, f32(_BIGF))
        words_ref[...] = w - oh * lsb
        return acc + idxf * (lane32 == j.astype(f32)).astype(f32)

    acc = lax.fori_loop(0, _NS, ext, jnp.zeros((_SB, _NS), f32))
    first = acc[:, 0:1]
    acc = jnp.where(acc == _BIGF, first, acc)
    acc = jnp.where(acc == _BIGF, 0.0, acc)
    o_ref[0] = acc.astype(i32) + b * _N


def _ball(xyzT, new_xyz, plo, phi, interpret=False):
    return pl.pallas_call(
        _ball_body,
        grid=(_B, _S // _SB),
        in_specs=[
            pl.BlockSpec((1, 3, _N), lambda b, s: (b, 0, 0)),
            pl.BlockSpec((1, _SB, 3), lambda b, s: (b, s, 0)),
            pl.BlockSpec((_CH, _CH // 32), lambda b, s: (0, 0)),
            pl.BlockSpec((_CH, _CH // 32), lambda b, s: (0, 0)),
        ],
        out_specs=pl.BlockSpec((1, _SB, _NS), lambda b, s: (b, s, 0)),
        out_shape=jax.ShapeDtypeStruct((_B, _S, _NS), jnp.int32),
        scratch_shapes=[pltpu.VMEM((_SB, _NW), jnp.int32)],
        interpret=interpret,
    )(xyzT, new_xyz, plo, phi)


def _pack_mats():
    n = jnp.arange(_CH, dtype=jnp.int32)
    w = jnp.arange(_CH // 32, dtype=jnp.int32)
    blk = (n[:, None] // 32) == w[None, :]
    bit = n % 32
    lo = jnp.where(blk & (bit[:, None] < 16), 2.0 ** bit.astype(jnp.float32)[:, None], 0.0)
    hi = jnp.where(blk & (bit[:, None] >= 16), 2.0 ** (bit - 16).astype(jnp.float32)[:, None], 0.0)
    return lo.astype(jnp.bfloat16), hi.astype(jnp.bfloat16)


_NIDX = _B * _S * _NS
_GW = 128


def _sc_gather(table, idx_flat):
    mesh = plsc.VectorSubcoreMesh(core_axis_name="core", subcore_axis_name="subcore")

    @pl.kernel(
        out_type=jax.ShapeDtypeStruct((_NIDX, _D), jnp.float32),
        mesh=mesh,
    )
    def gather_kernel(x_hbm, i_hbm, o_hbm):
        def body(i_vmem, o_vmem):
            pltpu.sync_copy(x_hbm.at[i_vmem.at[0]], o_vmem)

        pltpu.emit_pipeline(
            body,
            grid=(_NIDX // _GW,),
            in_specs=[pl.BlockSpec((1, _GW), index_map=lambda i: (0, i))],
            out_specs=[pl.BlockSpec((_GW, _D), index_map=lambda i: (i, 0))],
            core_axis_name=("core", "subcore"),
            dimension_semantics=(pltpu.PARALLEL,),
        )(i_hbm, o_hbm)

    return gather_kernel(table, idx_flat)


_SBM = 128
_COUT = 128


def _mlp_body(g_ref, nx_ref, w1_ref, g1_ref, b1_ref, w2_ref, g2_ref, b2_ref,
              w3_ref, g3_ref, b3_ref, o_ref):
    f32 = jnp.float32
    g = g_ref[0]
    g = g - nx_ref[0][:, None, :]
    a = g.reshape(_SBM * _NS, _D)
    dn = (((1,), (1,)), ((), ()))
    h = lax.dot_general(a, w1_ref[...], dn, preferred_element_type=f32)
    h = jnp.maximum(h * g1_ref[...] + b1_ref[...], 0.0)
    h = lax.dot_general(h, w2_ref[...], dn, preferred_element_type=f32)
    h = jnp.maximum(h * g2_ref[...] + b2_ref[...], 0.0)
    h = lax.dot_general(h, w3_ref[...], dn, preferred_element_type=f32)
    h = jnp.maximum(h * g3_ref[...] + b3_ref[...], 0.0)
    p = jnp.max(h.reshape(_SBM, _NS, _COUT), axis=1)
    o_ref[0] = p.T


def _mlp(gath, nxp, w1p, g1, b1, w2, g2, b2, w3, g3, b3, interpret=False):
    full = lambda shape: pl.BlockSpec(shape, lambda b, s: tuple(0 for _ in shape))
    return pl.pallas_call(
        _mlp_body,
        grid=(_B, _S // _SBM),
        in_specs=[
            pl.BlockSpec((1, _SBM, _NS, _D), lambda b, s: (b, s, 0, 0)),
            pl.BlockSpec((1, _SBM, _D), lambda b, s: (b, s, 0)),
            full((64, _D)), full((1, 64)), full((1, 64)),
            full((64, 64)), full((1, 64)), full((1, 64)),
            full((_COUT, 64)), full((1, _COUT)), full((1, _COUT)),
        ],
        out_specs=pl.BlockSpec((1, _COUT, _SBM), lambda b, s: (b, 0, s)),
        out_shape=jax.ShapeDtypeStruct((_B, _COUT, _S), jnp.float32),
        interpret=interpret,
    )(gath, nxp, w1p, g1, b1, w2, g2, b2, w3, g3, b3)


def kernel(xyz, features, W1, g1, b1, W2, g2, b2, W3, g3, b3):
    sample_inds, new_xyz = _fps(xyz)

    xyzT = jnp.transpose(xyz, (0, 2, 1))
    plo, phi = _pack_mats()
    idx = _ball(xyzT, new_xyz, plo, phi)

    pad = _D - 3 - _CIN
    table = jnp.concatenate(
        [xyz, features, jnp.zeros((_B, _N, pad), jnp.float32)], axis=-1
    ).reshape(_B * _N, _D)
    gath = _sc_gather(table, idx.reshape(1, _NIDX)).reshape(_B, _S, _NS, _D)

    nxp = jnp.concatenate(
        [new_xyz, jnp.zeros((_B, _S, _D - 3), jnp.float32)], axis=-1
    )
    w1p = jnp.concatenate([W1, jnp.zeros((64, pad), jnp.float32)], axis=-1)
    new_features = _mlp(
        gath, nxp, w1p,
        g1.reshape(1, 64), b1.reshape(1, 64),
        W2, g2.reshape(1, 64), b2.reshape(1, 64),
        W3, g3.reshape(1, _COUT), b3.reshape(1, _COUT),
    )
    return new_xyz, new_features, sample_inds


# speedup vs baseline: 32.5411x; 1.0344x over previous
import jax
import jax.numpy as jnp
from jax import lax
from jax.experimental import pallas as pl
from jax.experimental.pallas import tpu as pltpu
from jax.experimental.pallas import tpu_sc as plsc

_B, _N, _S, _NS = 4, 16384, 1024, 32
_R2 = 0.25
_CIN = 64
_D = 128
_ROWS = 8
_COLS = _N // _ROWS
_BIGF = float(_N)


def _fps_body(xs_ref, ys_ref, zs_ref, si_ref, nx_ref, dist_ref):
    f32 = jnp.float32
    shp = (_B, _ROWS, _COLS)
    iota = (
        lax.broadcasted_iota(jnp.int32, shp, 1) * _COLS
        + lax.broadcasted_iota(jnp.int32, shp, 2)
    ).astype(f32)
    dist_ref[...] = jnp.full(shp, 1e10, f32)

    def body(i, far):
        onehot = (iota == far).astype(f32)
        xs = xs_ref[...]
        ys = ys_ref[...]
        zs = zs_ref[...]
        cx = jnp.sum(jnp.sum(xs * onehot, axis=1, keepdims=True), axis=2, keepdims=True)
        cy = jnp.sum(jnp.sum(ys * onehot, axis=1, keepdims=True), axis=2, keepdims=True)
        cz = jnp.sum(jnp.sum(zs * onehot, axis=1, keepdims=True), axis=2, keepdims=True)
        si_ref[:, pl.ds(i, 1), :] = far.astype(jnp.int32)
        nx_ref[:, pl.ds(i, 1), :] = jnp.concatenate([cx, cy, cz], axis=2)
        dx = xs - cx
        dy = ys - cy
        dz = zs - cz
        d = (dx * dx + dy * dy) + dz * dz
        dmin = jnp.minimum(dist_ref[...], d)
        dist_ref[...] = dmin
        m = jnp.max(jnp.max(dmin, axis=1, keepdims=True), axis=2, keepdims=True)
        cand = jnp.where(dmin == m, iota, f32(3.0e10))
        far2 = jnp.min(jnp.min(cand, axis=1, keepdims=True), axis=2, keepdims=True)
        return far2

    lax.fori_loop(0, _S, body, jnp.zeros((_B, 1, 1), f32))


def _fps(xyz, interpret=False):
    xs = xyz[..., 0].reshape(_B, _ROWS, _COLS)
    ys = xyz[..., 1].reshape(_B, _ROWS, _COLS)
    zs = xyz[..., 2].reshape(_B, _ROWS, _COLS)
    si, nx = pl.pallas_call(
        _fps_body,
        out_shape=[
            jax.ShapeDtypeStruct((_B, _S, 1), jnp.int32),
            jax.ShapeDtypeStruct((_B, _S, 3), jnp.float32),
        ],
        scratch_shapes=[pltpu.VMEM((_B, _ROWS, _COLS), jnp.float32)],
        interpret=interpret,
    )(xs, ys, zs)
    return si[..., 0], nx


_SB = 64
_CH = 4096


_NW = _N // 32


def _ball_body(xt_ref, c_ref, plo_ref, phi_ref, o_ref):
    f32 = jnp.float32
    i32 = jnp.int32
    b = pl.program_id(0)
    c = c_ref[0]
    cx = c[:, 0:1]
    cy = c[:, 1:2]
    cz = c[:, 2:3]

    dn = (((1,), (0,)), ((), ()))
    wparts = []
    for k in range(_N // _CH):
        sl = pl.ds(k * _CH, _CH)
        xr = xt_ref[0, 0:1, sl]
        yr = xt_ref[0, 1:2, sl]
        zr = xt_ref[0, 2:3, sl]
        dx = xr - cx
        dy = yr - cy
        dz = zr - cz
        d = (dx * dx + dy * dy) + dz * dz
        bitsb = (d <= _R2).astype(jnp.bfloat16)
        wlo = lax.dot_general(bitsb, plo_ref[...], dn, preferred_element_type=f32)
        whi = lax.dot_general(bitsb, phi_ref[...], dn, preferred_element_type=f32)
        wparts.append(wlo.astype(i32) + (whi.astype(i32) << 16))
    words0 = jnp.concatenate(wparts, axis=1)

    lane32 = lax.broadcasted_iota(i32, (1, _NS), 1).astype(f32)
    wiota = lax.broadcasted_iota(i32, (1, _NW), 1).astype(f32)

    def ext(j, state):
        w, acc = state
        nz = w != 0
        cw = jnp.min(jnp.where(nz, wiota, f32(_NW)), axis=1, keepdims=True)
        oh = (wiota == cw).astype(i32)
        wsel = jnp.sum(w * oh, axis=1, keepdims=True)
        lsb = wsel & (-wsel)
        lf = jnp.abs(lsb.astype(f32))
        bit = (lax.bitcast_convert_type(lf, i32) >> 23) - 127
        idxf = cw * f32(32.0) + bit.astype(f32)
        idxf = jnp.where(cw < _NW, idxf, f32(_BIGF))
        return w - oh * lsb, acc + idxf * (lane32 == j.astype(f32)).astype(f32)

    _, acc = lax.fori_loop(
        0, _NS, ext, (words0, jnp.zeros((_SB, _NS), f32))
    )
    first = acc[:, 0:1]
    acc = jnp.where(acc == _BIGF, first, acc)
    acc = jnp.where(acc == _BIGF, 0.0, acc)
    o_ref[0] = acc.astype(i32) + b * _N


def _ball(xyzT, new_xyz, plo, phi, interpret=False):
    return pl.pallas_call(
        _ball_body,
        grid=(_B, _S // _SB),
        in_specs=[
            pl.BlockSpec((1, 3, _N), lambda b, s: (b, 0, 0)),
            pl.BlockSpec((1, _SB, 3), lambda b, s: (b, s, 0)),
            pl.BlockSpec((_CH, _CH // 32), lambda b, s: (0, 0)),
            pl.BlockSpec((_CH, _CH // 32), lambda b, s: (0, 0)),
        ],
        out_specs=pl.BlockSpec((1, _SB, _NS), lambda b, s: (b, s, 0)),
        out_shape=jax.ShapeDtypeStruct((_B, _S, _NS), jnp.int32),
        interpret=interpret,
    )(xyzT, new_xyz, plo, phi)


def _pack_mats():
    n = jnp.arange(_CH, dtype=jnp.int32)
    w = jnp.arange(_CH // 32, dtype=jnp.int32)
    blk = (n[:, None] // 32) == w[None, :]
    bit = n % 32
    lo = jnp.where(blk & (bit[:, None] < 16), 2.0 ** bit.astype(jnp.float32)[:, None], 0.0)
    hi = jnp.where(blk & (bit[:, None] >= 16), 2.0 ** (bit - 16).astype(jnp.float32)[:, None], 0.0)
    return lo.astype(jnp.bfloat16), hi.astype(jnp.bfloat16)


_NIDX = _B * _S * _NS
_GW = 128


def _sc_gather(table, idx_flat):
    mesh = plsc.VectorSubcoreMesh(core_axis_name="core", subcore_axis_name="subcore")

    @pl.kernel(
        out_type=jax.ShapeDtypeStruct((_NIDX, _D), jnp.float32),
        mesh=mesh,
    )
    def gather_kernel(x_hbm, i_hbm, o_hbm):
        def body(i_vmem, o_vmem):
            pltpu.sync_copy(x_hbm.at[i_vmem.at[0]], o_vmem)

        pltpu.emit_pipeline(
            body,
            grid=(_NIDX // _GW,),
            in_specs=[pl.BlockSpec((1, _GW), index_map=lambda i: (0, i))],
            out_specs=[pl.BlockSpec((_GW, _D), index_map=lambda i: (i, 0))],
            core_axis_name=("core", "subcore"),
            dimension_semantics=(pltpu.PARALLEL,),
        )(i_hbm, o_hbm)

    return gather_kernel(table, idx_flat)


_SBM = 128
_COUT = 128


def _mlp_body(g_ref, nx_ref, w1_ref, g1_ref, b1_ref, w2_ref, g2_ref, b2_ref,
              w3_ref, g3_ref, b3_ref, o_ref):
    f32 = jnp.float32
    g = g_ref[0]
    g = g - nx_ref[0][:, None, :]
    a = g.reshape(_SBM * _NS, _D)
    dn = (((1,), (1,)), ((), ()))
    h = lax.dot_general(a, w1_ref[...], dn, preferred_element_type=f32)
    h = jnp.maximum(h * g1_ref[...] + b1_ref[...], 0.0)
    h = lax.dot_general(h, w2_ref[...], dn, preferred_element_type=f32)
    h = jnp.maximum(h * g2_ref[...] + b2_ref[...], 0.0)
    h = lax.dot_general(h, w3_ref[...], dn, preferred_element_type=f32)
    h = jnp.maximum(h * g3_ref[...] + b3_ref[...], 0.0)
    p = jnp.max(h.reshape(_SBM, _NS, _COUT), axis=1)
    o_ref[0] = p.T


def _mlp(gath, nxp, w1p, g1, b1, w2, g2, b2, w3, g3, b3, interpret=False):
    full = lambda shape: pl.BlockSpec(shape, lambda b, s: tuple(0 for _ in shape))
    return pl.pallas_call(
        _mlp_body,
        grid=(_B, _S // _SBM),
        in_specs=[
            pl.BlockSpec((1, _SBM, _NS, _D), lambda b, s: (b, s, 0, 0)),
            pl.BlockSpec((1, _SBM, _D), lambda b, s: (b, s, 0)),
            full((64, _D)), full((1, 64)), full((1, 64)),
            full((64, 64)), full((1, 64)), full((1, 64)),
            full((_COUT, 64)), full((1, _COUT)), full((1, _COUT)),
        ],
        out_specs=pl.BlockSpec((1, _COUT, _SBM), lambda b, s: (b, 0, s)),
        out_shape=jax.ShapeDtypeStruct((_B, _COUT, _S), jnp.float32),
        interpret=interpret,
    )(gath, nxp, w1p, g1, b1, w2, g2, b2, w3, g3, b3)


def kernel(xyz, features, W1, g1, b1, W2, g2, b2, W3, g3, b3):
    sample_inds, new_xyz = _fps(xyz)

    xyzT = jnp.transpose(xyz, (0, 2, 1))
    plo, phi = _pack_mats()
    idx = _ball(xyzT, new_xyz, plo, phi)

    pad = _D - 3 - _CIN
    table = jnp.concatenate(
        [xyz, features, jnp.zeros((_B, _N, pad), jnp.float32)], axis=-1
    ).reshape(_B * _N, _D)
    gath = _sc_gather(table, idx.reshape(1, _NIDX)).reshape(_B, _S, _NS, _D)

    nxp = jnp.concatenate(
        [new_xyz, jnp.zeros((_B, _S, _D - 3), jnp.float32)], axis=-1
    )
    w1p = jnp.concatenate([W1, jnp.zeros((64, pad), jnp.float32)], axis=-1)
    new_features = _mlp(
        gath, nxp, w1p,
        g1.reshape(1, 64), b1.reshape(1, 64),
        W2, g2.reshape(1, 64), b2.reshape(1, 64),
        W3, g3.reshape(1, _COUT), b3.reshape(1, _COUT),
    )
    return new_xyz, new_features, sample_inds


# speedup vs baseline: 32.6166x; 1.0023x over previous
import jax
import jax.numpy as jnp
from jax import lax
from jax.experimental import pallas as pl
from jax.experimental.pallas import tpu as pltpu
from jax.experimental.pallas import tpu_sc as plsc

_B, _N, _S, _NS = 4, 16384, 1024, 32
_R2 = 0.25
_CIN = 64
_D = 128
_ROWS = 8
_COLS = _N // _ROWS
_BIGF = float(_N)


def _fps_body(xs_ref, ys_ref, zs_ref, si_ref, nx_ref, dist_ref):
    f32 = jnp.float32
    shp = (_B, _ROWS, _COLS)
    iota = (
        lax.broadcasted_iota(jnp.int32, shp, 1) * _COLS
        + lax.broadcasted_iota(jnp.int32, shp, 2)
    ).astype(f32)
    dist_ref[...] = jnp.full(shp, 1e10, f32)

    def body(i, far):
        onehot = (iota == far).astype(f32)
        xs = xs_ref[...]
        ys = ys_ref[...]
        zs = zs_ref[...]
        cx = jnp.sum(jnp.sum(xs * onehot, axis=1, keepdims=True), axis=2, keepdims=True)
        cy = jnp.sum(jnp.sum(ys * onehot, axis=1, keepdims=True), axis=2, keepdims=True)
        cz = jnp.sum(jnp.sum(zs * onehot, axis=1, keepdims=True), axis=2, keepdims=True)
        si_ref[:, pl.ds(i, 1), :] = far.astype(jnp.int32)
        nx_ref[:, pl.ds(i, 1), :] = jnp.concatenate([cx, cy, cz], axis=2)
        dx = xs - cx
        dy = ys - cy
        dz = zs - cz
        d = (dx * dx + dz * dz) + dy * dy
        dmin = jnp.minimum(dist_ref[...], d)
        dist_ref[...] = dmin
        m = jnp.max(jnp.max(dmin, axis=1, keepdims=True), axis=2, keepdims=True)
        cand = jnp.where(dmin == m, iota, f32(3.0e10))
        far2 = jnp.min(jnp.min(cand, axis=1, keepdims=True), axis=2, keepdims=True)
        return far2

    lax.fori_loop(0, _S, body, jnp.zeros((_B, 1, 1), f32))


def _fps(xyz, interpret=False):
    xs = xyz[..., 0].reshape(_B, _ROWS, _COLS)
    ys = xyz[..., 1].reshape(_B, _ROWS, _COLS)
    zs = xyz[..., 2].reshape(_B, _ROWS, _COLS)
    si, nx = pl.pallas_call(
        _fps_body,
        out_shape=[
            jax.ShapeDtypeStruct((_B, _S, 1), jnp.int32),
            jax.ShapeDtypeStruct((_B, _S, 3), jnp.float32),
        ],
        scratch_shapes=[pltpu.VMEM((_B, _ROWS, _COLS), jnp.float32)],
        interpret=interpret,
    )(xs, ys, zs)
    return si[..., 0], nx


_SB = 64
_CH = 4096


_NW = _N // 32


def _ball_body(xt_ref, c_ref, plo_ref, phi_ref, o_ref):
    f32 = jnp.float32
    i32 = jnp.int32
    b = pl.program_id(0)
    c = c_ref[0]
    cx = c[:, 0:1]
    cy = c[:, 1:2]
    cz = c[:, 2:3]

    dn = (((1,), (0,)), ((), ()))
    wparts = []
    for k in range(_N // _CH):
        sl = pl.ds(k * _CH, _CH)
        xr = xt_ref[0, 0:1, sl]
        yr = xt_ref[0, 1:2, sl]
        zr = xt_ref[0, 2:3, sl]
        dx = xr - cx
        dy = yr - cy
        dz = zr - cz
        d = (dx * dx + dz * dz) + dy * dy
        bitsb = (d <= _R2).astype(jnp.bfloat16)
        wlo = lax.dot_general(bitsb, plo_ref[...], dn, preferred_element_type=f32)
        whi = lax.dot_general(bitsb, phi_ref[...], dn, preferred_element_type=f32)
        wparts.append(wlo.astype(i32) + (whi.astype(i32) << 16))
    words0 = jnp.concatenate(wparts, axis=1)

    lane32 = lax.broadcasted_iota(i32, (1, _NS), 1).astype(f32)
    wiota = lax.broadcasted_iota(i32, (1, _NW), 1).astype(f32)

    def ext(j, state):
        w, acc = state
        nz = w != 0
        cw = jnp.min(jnp.where(nz, wiota, f32(_NW)), axis=1, keepdims=True)
        oh = (wiota == cw).astype(i32)
        wsel = jnp.sum(w * oh, axis=1, keepdims=True)
        lsb = wsel & (-wsel)
        lf = jnp.abs(lsb.astype(f32))
        bit = (lax.bitcast_convert_type(lf, i32) >> 23) - 127
        idxf = cw * f32(32.0) + bit.astype(f32)
        idxf = jnp.where(cw < _NW, idxf, f32(_BIGF))
        return w - oh * lsb, acc + idxf * (lane32 == j.astype(f32)).astype(f32)

    _, acc = lax.fori_loop(
        0, _NS, ext, (words0, jnp.zeros((_SB, _NS), f32))
    )
    first = acc[:, 0:1]
    acc = jnp.where(acc == _BIGF, first, acc)
    acc = jnp.where(acc == _BIGF, 0.0, acc)
    o_ref[0] = acc.astype(i32) + b * _N


def _ball(xyzT, new_xyz, plo, phi, interpret=False):
    return pl.pallas_call(
        _ball_body,
        grid=(_B, _S // _SB),
        in_specs=[
            pl.BlockSpec((1, 3, _N), lambda b, s: (b, 0, 0)),
            pl.BlockSpec((1, _SB, 3), lambda b, s: (b, s, 0)),
            pl.BlockSpec((_CH, _CH // 32), lambda b, s: (0, 0)),
            pl.BlockSpec((_CH, _CH // 32), lambda b, s: (0, 0)),
        ],
        out_specs=pl.BlockSpec((1, _SB, _NS), lambda b, s: (b, s, 0)),
        out_shape=jax.ShapeDtypeStruct((_B, _S, _NS), jnp.int32),
        interpret=interpret,
    )(xyzT, new_xyz, plo, phi)


def _pack_mats():
    n = jnp.arange(_CH, dtype=jnp.int32)
    w = jnp.arange(_CH // 32, dtype=jnp.int32)
    blk = (n[:, None] // 32) == w[None, :]
    bit = n % 32
    lo = jnp.where(blk & (bit[:, None] < 16), 2.0 ** bit.astype(jnp.float32)[:, None], 0.0)
    hi = jnp.where(blk & (bit[:, None] >= 16), 2.0 ** (bit - 16).astype(jnp.float32)[:, None], 0.0)
    return lo.astype(jnp.bfloat16), hi.astype(jnp.bfloat16)


_NIDX = _B * _S * _NS
_GW = 128


def _sc_gather(table, idx_flat):
    mesh = plsc.VectorSubcoreMesh(core_axis_name="core", subcore_axis_name="subcore")

    @pl.kernel(
        out_type=jax.ShapeDtypeStruct((_NIDX, _D), jnp.float32),
        mesh=mesh,
    )
    def gather_kernel(x_hbm, i_hbm, o_hbm):
        def body(i_vmem, o_vmem):
            pltpu.sync_copy(x_hbm.at[i_vmem.at[0]], o_vmem)

        pltpu.emit_pipeline(
            body,
            grid=(_NIDX // _GW,),
            in_specs=[pl.BlockSpec((1, _GW), index_map=lambda i: (0, i))],
            out_specs=[pl.BlockSpec((_GW, _D), index_map=lambda i: (i, 0))],
            core_axis_name=("core", "subcore"),
            dimension_semantics=(pltpu.PARALLEL,),
        )(i_hbm, o_hbm)

    return gather_kernel(table, idx_flat)


_SBM = 128
_COUT = 128


def _mlp_body(g_ref, nx_ref, w1_ref, g1_ref, b1_ref, w2_ref, g2_ref, b2_ref,
              w3_ref, g3_ref, b3_ref, o_ref):
    f32 = jnp.float32
    g = g_ref[0]
    g = g - nx_ref[0][:, None, :]
    a = g.reshape(_SBM * _NS, _D)
    dn = (((1,), (1,)), ((), ()))
    h = lax.dot_general(a, w1_ref[...], dn, preferred_element_type=f32)
    h = jnp.maximum(h * g1_ref[...] + b1_ref[...], 0.0)
    h = lax.dot_general(h, w2_ref[...], dn, preferred_element_type=f32)
    h = jnp.maximum(h * g2_ref[...] + b2_ref[...], 0.0)
    h = lax.dot_general(h, w3_ref[...], dn, preferred_element_type=f32)
    h = jnp.maximum(h * g3_ref[...] + b3_ref[...], 0.0)
    p = jnp.max(h.reshape(_SBM, _NS, _COUT), axis=1)
    o_ref[0] = p.T


def _mlp(gath, nxp, w1p, g1, b1, w2, g2, b2, w3, g3, b3, interpret=False):
    full = lambda shape: pl.BlockSpec(shape, lambda b, s: tuple(0 for _ in shape))
    return pl.pallas_call(
        _mlp_body,
        grid=(_B, _S // _SBM),
        in_specs=[
            pl.BlockSpec((1, _SBM, _NS, _D), lambda b, s: (b, s, 0, 0)),
            pl.BlockSpec((1, _SBM, _D), lambda b, s: (b, s, 0)),
            full((64, _D)), full((1, 64)), full((1, 64)),
            full((64, 64)), full((1, 64)), full((1, 64)),
            full((_COUT, 64)), full((1, _COUT)), full((1, _COUT)),
        ],
        out_specs=pl.BlockSpec((1, _COUT, _SBM), lambda b, s: (b, 0, s)),
        out_shape=jax.ShapeDtypeStruct((_B, _COUT, _S), jnp.float32),
        interpret=interpret,
    )(gath, nxp, w1p, g1, b1, w2, g2, b2, w3, g3, b3)


def kernel(xyz, features, W1, g1, b1, W2, g2, b2, W3, g3, b3):
    sample_inds, new_xyz = _fps(xyz)

    xyzT = jnp.transpose(xyz, (0, 2, 1))
    plo, phi = _pack_mats()
    idx = _ball(xyzT, new_xyz, plo, phi)

    pad = _D - 3 - _CIN
    table = jnp.concatenate(
        [xyz, features, jnp.zeros((_B, _N, pad), jnp.float32)], axis=-1
    ).reshape(_B * _N, _D)
    gath = _sc_gather(table, idx.reshape(1, _NIDX)).reshape(_B, _S, _NS, _D)

    nxp = jnp.concatenate(
        [new_xyz, jnp.zeros((_B, _S, _D - 3), jnp.float32)], axis=-1
    )
    w1p = jnp.concatenate([W1, jnp.zeros((64, pad), jnp.float32)], axis=-1)
    new_features = _mlp(
        gath, nxp, w1p,
        g1.reshape(1, 64), b1.reshape(1, 64),
        W2, g2.reshape(1, 64), b2.reshape(1, 64),
        W3, g3.reshape(1, _COUT), b3.reshape(1, _COUT),
    )
    return new_xyz, new_features, sample_inds


# speedup vs baseline: 36.5127x; 1.1195x over previous
import jax
import jax.numpy as jnp
from jax import lax
from jax.experimental import pallas as pl
from jax.experimental.pallas import tpu as pltpu
from jax.experimental.pallas import tpu_sc as plsc

_B, _N, _S, _NS = 4, 16384, 1024, 32
_R2 = 0.25
_CIN = 64
_D = 128
_ROWS = 8
_COLS = _N // _ROWS
_BIGF = float(_N)


def _fps_body(xs_ref, ys_ref, zs_ref, si_ref, nx_ref, dist_ref):
    f32 = jnp.float32
    shp = (_B, _ROWS, _COLS)
    iota = (
        lax.broadcasted_iota(jnp.int32, shp, 1) * _COLS
        + lax.broadcasted_iota(jnp.int32, shp, 2)
    ).astype(f32)
    dist_ref[...] = jnp.full(shp, 1e10, f32)

    def body(i, far):
        onehot = (iota == far).astype(f32)
        xs = xs_ref[...]
        ys = ys_ref[...]
        zs = zs_ref[...]
        cx = jnp.sum(jnp.sum(xs * onehot, axis=1, keepdims=True), axis=2, keepdims=True)
        cy = jnp.sum(jnp.sum(ys * onehot, axis=1, keepdims=True), axis=2, keepdims=True)
        cz = jnp.sum(jnp.sum(zs * onehot, axis=1, keepdims=True), axis=2, keepdims=True)
        si_ref[:, pl.ds(i, 1), :] = far.astype(jnp.int32)
        nx_ref[:, pl.ds(i, 1), :] = jnp.concatenate([cx, cy, cz], axis=2)
        dx = xs - cx
        dy = ys - cy
        dz = zs - cz
        d = (dx * dx + dz * dz) + dy * dy
        dmin = jnp.minimum(dist_ref[...], d)
        dist_ref[...] = dmin
        m = jnp.max(jnp.max(dmin, axis=1, keepdims=True), axis=2, keepdims=True)
        cand = jnp.where(dmin == m, iota, f32(3.0e10))
        far2 = jnp.min(jnp.min(cand, axis=1, keepdims=True), axis=2, keepdims=True)
        return far2

    lax.fori_loop(0, _S, body, jnp.zeros((_B, 1, 1), f32))


def _fps(xyz, interpret=False):
    xs = xyz[..., 0].reshape(_B, _ROWS, _COLS)
    ys = xyz[..., 1].reshape(_B, _ROWS, _COLS)
    zs = xyz[..., 2].reshape(_B, _ROWS, _COLS)
    si, nx = pl.pallas_call(
        _fps_body,
        out_shape=[
            jax.ShapeDtypeStruct((_B, _S, 1), jnp.int32),
            jax.ShapeDtypeStruct((_B, _S, 3), jnp.float32),
        ],
        scratch_shapes=[pltpu.VMEM((_B, _ROWS, _COLS), jnp.float32)],
        interpret=interpret,
    )(xs, ys, zs)
    return si[..., 0], nx


_SB = 128
_CH = 4096


_NW = _N // 32


def _ball_body(xt_ref, c_ref, plo_ref, phi_ref, o_ref):
    f32 = jnp.float32
    i32 = jnp.int32
    b = pl.program_id(0)
    c = c_ref[0]
    cx = c[:, 0:1]
    cy = c[:, 1:2]
    cz = c[:, 2:3]

    dn = (((1,), (0,)), ((), ()))
    wparts = []
    for k in range(_N // _CH):
        sl = pl.ds(k * _CH, _CH)
        xr = xt_ref[0, 0:1, sl]
        yr = xt_ref[0, 1:2, sl]
        zr = xt_ref[0, 2:3, sl]
        dx = xr - cx
        dy = yr - cy
        dz = zr - cz
        d = (dx * dx + dz * dz) + dy * dy
        bitsb = (d <= _R2).astype(jnp.bfloat16)
        wlo = lax.dot_general(bitsb, plo_ref[...], dn, preferred_element_type=f32)
        whi = lax.dot_general(bitsb, phi_ref[...], dn, preferred_element_type=f32)
        wparts.append(wlo.astype(i32) + (whi.astype(i32) << 16))
    words0 = jnp.concatenate(wparts, axis=1)

    lane32 = lax.broadcasted_iota(i32, (1, _NS), 1).astype(f32)
    wiota = lax.broadcasted_iota(i32, (1, _NW), 1).astype(f32)

    def ext(j, state):
        w, acc = state
        nz = w != 0
        cw = jnp.min(jnp.where(nz, wiota, f32(_NW)), axis=1, keepdims=True)
        oh = (wiota == cw).astype(i32)
        wsel = jnp.sum(w * oh, axis=1, keepdims=True)
        lsb = wsel & (-wsel)
        lf = jnp.abs(lsb.astype(f32))
        bit = (lax.bitcast_convert_type(lf, i32) >> 23) - 127
        idxf = cw * f32(32.0) + bit.astype(f32)
        idxf = jnp.where(cw < _NW, idxf, f32(_BIGF))
        return w - oh * lsb, acc + idxf * (lane32 == j.astype(f32)).astype(f32)

    _, acc = lax.fori_loop(
        0, _NS, ext, (words0, jnp.zeros((_SB, _NS), f32))
    )
    first = acc[:, 0:1]
    acc = jnp.where(acc == _BIGF, first, acc)
    acc = jnp.where(acc == _BIGF, 0.0, acc)
    o_ref[0] = acc.astype(i32) + b * _N


def _ball(xyzT, new_xyz, plo, phi, interpret=False):
    return pl.pallas_call(
        _ball_body,
        grid=(_B, _S // _SB),
        in_specs=[
            pl.BlockSpec((1, 3, _N), lambda b, s: (b, 0, 0)),
            pl.BlockSpec((1, _SB, 3), lambda b, s: (b, s, 0)),
            pl.BlockSpec((_CH, _CH // 32), lambda b, s: (0, 0)),
            pl.BlockSpec((_CH, _CH // 32), lambda b, s: (0, 0)),
        ],
        out_specs=pl.BlockSpec((1, _SB, _NS), lambda b, s: (b, s, 0)),
        out_shape=jax.ShapeDtypeStruct((_B, _S, _NS), jnp.int32),
        interpret=interpret,
    )(xyzT, new_xyz, plo, phi)


def _pack_mats():
    n = jnp.arange(_CH, dtype=jnp.int32)
    w = jnp.arange(_CH // 32, dtype=jnp.int32)
    blk = (n[:, None] // 32) == w[None, :]
    bit = n % 32
    lo = jnp.where(blk & (bit[:, None] < 16), 2.0 ** bit.astype(jnp.float32)[:, None], 0.0)
    hi = jnp.where(blk & (bit[:, None] >= 16), 2.0 ** (bit - 16).astype(jnp.float32)[:, None], 0.0)
    return lo.astype(jnp.bfloat16), hi.astype(jnp.bfloat16)


_NIDX = _B * _S * _NS
_GW = 128


def _sc_gather(table, idx_flat):
    mesh = plsc.VectorSubcoreMesh(core_axis_name="core", subcore_axis_name="subcore")

    @pl.kernel(
        out_type=jax.ShapeDtypeStruct((_NIDX, _D), jnp.float32),
        mesh=mesh,
    )
    def gather_kernel(x_hbm, i_hbm, o_hbm):
        def body(i_vmem, o_vmem):
            pltpu.sync_copy(x_hbm.at[i_vmem.at[0]], o_vmem)

        pltpu.emit_pipeline(
            body,
            grid=(_NIDX // _GW,),
            in_specs=[pl.BlockSpec((1, _GW), index_map=lambda i: (0, i))],
            out_specs=[pl.BlockSpec((_GW, _D), index_map=lambda i: (i, 0))],
            core_axis_name=("core", "subcore"),
            dimension_semantics=(pltpu.PARALLEL,),
        )(i_hbm, o_hbm)

    return gather_kernel(table, idx_flat)


_SBM = 128
_COUT = 128


def _mlp_body(g_ref, nx_ref, w1_ref, g1_ref, b1_ref, w2_ref, g2_ref, b2_ref,
              w3_ref, g3_ref, b3_ref, o_ref):
    f32 = jnp.float32
    g = g_ref[0]
    g = g - nx_ref[0][:, None, :]
    a = g.reshape(_SBM * _NS, _D)
    dn = (((1,), (1,)), ((), ()))
    h = lax.dot_general(a, w1_ref[...], dn, preferred_element_type=f32)
    h = jnp.maximum(h * g1_ref[...] + b1_ref[...], 0.0)
    h = lax.dot_general(h, w2_ref[...], dn, preferred_element_type=f32)
    h = jnp.maximum(h * g2_ref[...] + b2_ref[...], 0.0)
    h = lax.dot_general(h, w3_ref[...], dn, preferred_element_type=f32)
    h = jnp.maximum(h * g3_ref[...] + b3_ref[...], 0.0)
    p = jnp.max(h.reshape(_SBM, _NS, _COUT), axis=1)
    o_ref[0] = p.T


def _mlp(gath, nxp, w1p, g1, b1, w2, g2, b2, w3, g3, b3, interpret=False):
    full = lambda shape: pl.BlockSpec(shape, lambda b, s: tuple(0 for _ in shape))
    return pl.pallas_call(
        _mlp_body,
        grid=(_B, _S // _SBM),
        in_specs=[
            pl.BlockSpec((1, _SBM, _NS, _D), lambda b, s: (b, s, 0, 0)),
            pl.BlockSpec((1, _SBM, _D), lambda b, s: (b, s, 0)),
            full((64, _D)), full((1, 64)), full((1, 64)),
            full((64, 64)), full((1, 64)), full((1, 64)),
            full((_COUT, 64)), full((1, _COUT)), full((1, _COUT)),
        ],
        out_specs=pl.BlockSpec((1, _COUT, _SBM), lambda b, s: (b, 0, s)),
        out_shape=jax.ShapeDtypeStruct((_B, _COUT, _S), jnp.float32),
        interpret=interpret,
    )(gath, nxp, w1p, g1, b1, w2, g2, b2, w3, g3, b3)


def kernel(xyz, features, W1, g1, b1, W2, g2, b2, W3, g3, b3):
    sample_inds, new_xyz = _fps(xyz)

    xyzT = jnp.transpose(xyz, (0, 2, 1))
    plo, phi = _pack_mats()
    idx = _ball(xyzT, new_xyz, plo, phi)

    pad = _D - 3 - _CIN
    table = jnp.concatenate(
        [xyz, features, jnp.zeros((_B, _N, pad), jnp.float32)], axis=-1
    ).reshape(_B * _N, _D)
    gath = _sc_gather(table, idx.reshape(1, _NIDX)).reshape(_B, _S, _NS, _D)

    nxp = jnp.concatenate(
        [new_xyz, jnp.zeros((_B, _S, _D - 3), jnp.float32)], axis=-1
    )
    w1p = jnp.concatenate([W1, jnp.zeros((64, pad), jnp.float32)], axis=-1)
    new_features = _mlp(
        gath, nxp, w1p,
        g1.reshape(1, 64), b1.reshape(1, 64),
        W2, g2.reshape(1, 64), b2.reshape(1, 64),
        W3, g3.reshape(1, _COUT), b3.reshape(1, _COUT),
    )
    return new_xyz, new_features, sample_inds


# speedup vs baseline: 39.4055x; 1.0792x over previous
import jax
import jax.numpy as jnp
from jax import lax
from jax.experimental import pallas as pl
from jax.experimental.pallas import tpu as pltpu
from jax.experimental.pallas import tpu_sc as plsc

_B, _N, _S, _NS = 4, 16384, 1024, 32
_R2 = 0.25
_CIN = 64
_D = 128
_ROWS = 8
_COLS = _N // _ROWS
_BIGF = float(_N)


def _fps_body(xs_ref, ys_ref, zs_ref, si_ref, nx_ref, dist_ref):
    f32 = jnp.float32
    shp = (_B, _ROWS, _COLS)
    iota = (
        lax.broadcasted_iota(jnp.int32, shp, 1) * _COLS
        + lax.broadcasted_iota(jnp.int32, shp, 2)
    ).astype(f32)
    dist_ref[...] = jnp.full(shp, 1e10, f32)

    def body(i, far):
        onehot = (iota == far).astype(f32)
        xs = xs_ref[...]
        ys = ys_ref[...]
        zs = zs_ref[...]
        cx = jnp.sum(jnp.sum(xs * onehot, axis=1, keepdims=True), axis=2, keepdims=True)
        cy = jnp.sum(jnp.sum(ys * onehot, axis=1, keepdims=True), axis=2, keepdims=True)
        cz = jnp.sum(jnp.sum(zs * onehot, axis=1, keepdims=True), axis=2, keepdims=True)
        si_ref[:, pl.ds(i, 1), :] = far.astype(jnp.int32)
        nx_ref[:, pl.ds(i, 1), :] = jnp.concatenate([cx, cy, cz], axis=2)
        dx = xs - cx
        dy = ys - cy
        dz = zs - cz
        d = (dx * dx + dz * dz) + dy * dy
        dmin = jnp.minimum(dist_ref[...], d)
        dist_ref[...] = dmin
        m = jnp.max(jnp.max(dmin, axis=1, keepdims=True), axis=2, keepdims=True)
        cand = jnp.where(dmin == m, iota, f32(3.0e10))
        far2 = jnp.min(jnp.min(cand, axis=1, keepdims=True), axis=2, keepdims=True)
        return far2

    lax.fori_loop(0, _S, body, jnp.zeros((_B, 1, 1), f32))


def _fps(xyz, interpret=False):
    xs = xyz[..., 0].reshape(_B, _ROWS, _COLS)
    ys = xyz[..., 1].reshape(_B, _ROWS, _COLS)
    zs = xyz[..., 2].reshape(_B, _ROWS, _COLS)
    si, nx = pl.pallas_call(
        _fps_body,
        out_shape=[
            jax.ShapeDtypeStruct((_B, _S, 1), jnp.int32),
            jax.ShapeDtypeStruct((_B, _S, 3), jnp.float32),
        ],
        scratch_shapes=[pltpu.VMEM((_B, _ROWS, _COLS), jnp.float32)],
        interpret=interpret,
    )(xs, ys, zs)
    return si[..., 0], nx


_SB = 256
_CH = 4096


_NW = _N // 32


def _ball_body(xt_ref, c_ref, plo_ref, phi_ref, o_ref):
    f32 = jnp.float32
    i32 = jnp.int32
    b = pl.program_id(0)
    c = c_ref[0]
    cx = c[:, 0:1]
    cy = c[:, 1:2]
    cz = c[:, 2:3]

    dn = (((1,), (0,)), ((), ()))
    wparts = []
    for k in range(_N // _CH):
        sl = pl.ds(k * _CH, _CH)
        xr = xt_ref[0, 0:1, sl]
        yr = xt_ref[0, 1:2, sl]
        zr = xt_ref[0, 2:3, sl]
        dx = xr - cx
        dy = yr - cy
        dz = zr - cz
        d = (dx * dx + dz * dz) + dy * dy
        bitsb = (d <= _R2).astype(jnp.bfloat16)
        wlo = lax.dot_general(bitsb, plo_ref[...], dn, preferred_element_type=f32)
        whi = lax.dot_general(bitsb, phi_ref[...], dn, preferred_element_type=f32)
        wparts.append(wlo.astype(i32) + (whi.astype(i32) << 16))
    words0 = jnp.concatenate(wparts, axis=1)

    lane32 = lax.broadcasted_iota(i32, (1, _NS), 1).astype(f32)
    wiota = lax.broadcasted_iota(i32, (1, _NW), 1).astype(f32)

    def ext(j, state):
        w, acc = state
        nz = w != 0
        cw = jnp.min(jnp.where(nz, wiota, f32(_NW)), axis=1, keepdims=True)
        oh = (wiota == cw).astype(i32)
        wsel = jnp.sum(w * oh, axis=1, keepdims=True)
        lsb = wsel & (-wsel)
        lf = jnp.abs(lsb.astype(f32))
        bit = (lax.bitcast_convert_type(lf, i32) >> 23) - 127
        idxf = cw * f32(32.0) + bit.astype(f32)
        idxf = jnp.where(cw < _NW, idxf, f32(_BIGF))
        return w - oh * lsb, acc + idxf * (lane32 == j.astype(f32)).astype(f32)

    _, acc = lax.fori_loop(
        0, _NS, ext, (words0, jnp.zeros((_SB, _NS), f32))
    )
    first = acc[:, 0:1]
    acc = jnp.where(acc == _BIGF, first, acc)
    acc = jnp.where(acc == _BIGF, 0.0, acc)
    o_ref[0] = acc.astype(i32) + b * _N


def _ball(xyzT, new_xyz, plo, phi, interpret=False):
    return pl.pallas_call(
        _ball_body,
        grid=(_B, _S // _SB),
        in_specs=[
            pl.BlockSpec((1, 3, _N), lambda b, s: (b, 0, 0)),
            pl.BlockSpec((1, _SB, 3), lambda b, s: (b, s, 0)),
            pl.BlockSpec((_CH, _CH // 32), lambda b, s: (0, 0)),
            pl.BlockSpec((_CH, _CH // 32), lambda b, s: (0, 0)),
        ],
        out_specs=pl.BlockSpec((1, _SB, _NS), lambda b, s: (b, s, 0)),
        out_shape=jax.ShapeDtypeStruct((_B, _S, _NS), jnp.int32),
        interpret=interpret,
    )(xyzT, new_xyz, plo, phi)


def _pack_mats():
    n = jnp.arange(_CH, dtype=jnp.int32)
    w = jnp.arange(_CH // 32, dtype=jnp.int32)
    blk = (n[:, None] // 32) == w[None, :]
    bit = n % 32
    lo = jnp.where(blk & (bit[:, None] < 16), 2.0 ** bit.astype(jnp.float32)[:, None], 0.0)
    hi = jnp.where(blk & (bit[:, None] >= 16), 2.0 ** (bit - 16).astype(jnp.float32)[:, None], 0.0)
    return lo.astype(jnp.bfloat16), hi.astype(jnp.bfloat16)


_NIDX = _B * _S * _NS
_GW = 128


def _sc_gather(table, idx_flat):
    mesh = plsc.VectorSubcoreMesh(core_axis_name="core", subcore_axis_name="subcore")

    @pl.kernel(
        out_type=jax.ShapeDtypeStruct((_NIDX, _D), jnp.float32),
        mesh=mesh,
    )
    def gather_kernel(x_hbm, i_hbm, o_hbm):
        def body(i_vmem, o_vmem):
            pltpu.sync_copy(x_hbm.at[i_vmem.at[0]], o_vmem)

        pltpu.emit_pipeline(
            body,
            grid=(_NIDX // _GW,),
            in_specs=[pl.BlockSpec((1, _GW), index_map=lambda i: (0, i))],
            out_specs=[pl.BlockSpec((_GW, _D), index_map=lambda i: (i, 0))],
            core_axis_name=("core", "subcore"),
            dimension_semantics=(pltpu.PARALLEL,),
        )(i_hbm, o_hbm)

    return gather_kernel(table, idx_flat)


_SBM = 128
_COUT = 128


def _mlp_body(g_ref, nx_ref, w1_ref, g1_ref, b1_ref, w2_ref, g2_ref, b2_ref,
              w3_ref, g3_ref, b3_ref, o_ref):
    f32 = jnp.float32
    g = g_ref[0]
    g = g - nx_ref[0][:, None, :]
    a = g.reshape(_SBM * _NS, _D)
    dn = (((1,), (1,)), ((), ()))
    h = lax.dot_general(a, w1_ref[...], dn, preferred_element_type=f32)
    h = jnp.maximum(h * g1_ref[...] + b1_ref[...], 0.0)
    h = lax.dot_general(h, w2_ref[...], dn, preferred_element_type=f32)
    h = jnp.maximum(h * g2_ref[...] + b2_ref[...], 0.0)
    h = lax.dot_general(h, w3_ref[...], dn, preferred_element_type=f32)
    h = jnp.maximum(h * g3_ref[...] + b3_ref[...], 0.0)
    p = jnp.max(h.reshape(_SBM, _NS, _COUT), axis=1)
    o_ref[0] = p.T


def _mlp(gath, nxp, w1p, g1, b1, w2, g2, b2, w3, g3, b3, interpret=False):
    full = lambda shape: pl.BlockSpec(shape, lambda b, s: tuple(0 for _ in shape))
    return pl.pallas_call(
        _mlp_body,
        grid=(_B, _S // _SBM),
        in_specs=[
            pl.BlockSpec((1, _SBM, _NS, _D), lambda b, s: (b, s, 0, 0)),
            pl.BlockSpec((1, _SBM, _D), lambda b, s: (b, s, 0)),
            full((64, _D)), full((1, 64)), full((1, 64)),
            full((64, 64)), full((1, 64)), full((1, 64)),
            full((_COUT, 64)), full((1, _COUT)), full((1, _COUT)),
        ],
        out_specs=pl.BlockSpec((1, _COUT, _SBM), lambda b, s: (b, 0, s)),
        out_shape=jax.ShapeDtypeStruct((_B, _COUT, _S), jnp.float32),
        interpret=interpret,
    )(gath, nxp, w1p, g1, b1, w2, g2, b2, w3, g3, b3)


def kernel(xyz, features, W1, g1, b1, W2, g2, b2, W3, g3, b3):
    sample_inds, new_xyz = _fps(xyz)

    xyzT = jnp.transpose(xyz, (0, 2, 1))
    plo, phi = _pack_mats()
    idx = _ball(xyzT, new_xyz, plo, phi)

    pad = _D - 3 - _CIN
    table = jnp.concatenate(
        [xyz, features, jnp.zeros((_B, _N, pad), jnp.float32)], axis=-1
    ).reshape(_B * _N, _D)
    gath = _sc_gather(table, idx.reshape(1, _NIDX)).reshape(_B, _S, _NS, _D)

    nxp = jnp.concatenate(
        [new_xyz, jnp.zeros((_B, _S, _D - 3), jnp.float32)], axis=-1
    )
    w1p = jnp.concatenate([W1, jnp.zeros((64, pad), jnp.float32)], axis=-1)
    new_features = _mlp(
        gath, nxp, w1p,
        g1.reshape(1, 64), b1.reshape(1, 64),
        W2, g2.reshape(1, 64), b2.reshape(1, 64),
        W3, g3.reshape(1, _COUT), b3.reshape(1, _COUT),
    )
    return new_xyz, new_features, sample_inds
